# baseline (speedup 1.0000x reference)
import functools

import jax
import jax.numpy as jnp
from jax import lax
from jax.experimental import pallas as pl
from jax.experimental.pallas import tpu as pltpu
from jax.experimental.pallas import tpu_sc as plsc

B = 1024
T = 200
D = 64
BT = B * T
NC = 2
NS = 16
NW = NC * NS
B_PER_W = BT // NW
CHUNK = 40
N_CHUNKS = B_PER_W // CHUNK
POS_CYCLE = T // CHUNK
LANES = 16
GROUPS = D // LANES

_mesh = plsc.VectorSubcoreMesh(core_axis_name="c", subcore_axis_name="s")


@functools.partial(
    pl.kernel,
    mesh=_mesh,
    out_type=jax.ShapeDtypeStruct((BT, D), jnp.float32),
    scratch_types=[
        pltpu.VMEM((N_CHUNKS, CHUNK), jnp.int32),
        pltpu.VMEM((CHUNK, D), jnp.float32),
        pltpu.VMEM((T, D), jnp.float32),
        pltpu.SemaphoreType.DMA,
    ],
    compiler_params=pltpu.CompilerParams(use_tc_tiling_on_sc=False),
)
def _embed_sc(x_hbm, tok_hbm, pos_hbm, out_hbm, idx_v, rows_v, pos_v, gsem):
    cid = lax.axis_index("c")
    sid = lax.axis_index("s")
    wid = sid * NC + cid
    base = wid * B_PER_W

    pltpu.sync_copy(x_hbm.at[wid], idx_v)
    pltpu.sync_copy(pos_hbm.at[pl.ds(0, T)], pos_v)

    def chunk_body(ch, carry):
        pltpu.async_copy(tok_hbm.at[idx_v.at[ch]], rows_v, gsem).wait()
        p0 = lax.rem(ch, POS_CYCLE) * CHUNK

        def row_body(r, cr):
            for g in range(GROUPS):
                sl = pl.ds(g * LANES, LANES)
                rows_v[r, sl] = rows_v[r, sl] + pos_v[p0 + r, sl]
            return cr

        lax.fori_loop(0, CHUNK, row_body, 0)
        pltpu.sync_copy(rows_v, out_hbm.at[pl.ds(base + ch * CHUNK, CHUNK)])
        return carry

    lax.fori_loop(0, N_CHUNKS, chunk_body, 0)


def kernel(x, token_emb, pos_emb):
    xw = x.reshape(NW, N_CHUNKS, CHUNK).astype(jnp.int32)
    out = _embed_sc(xw, token_emb, pos_emb)
    return out.reshape(B, T, D)


# speedup vs baseline: 1.2795x; 1.2795x over previous
import functools

import jax
import jax.numpy as jnp
from jax import lax
from jax.experimental import pallas as pl
from jax.experimental.pallas import tpu as pltpu
from jax.experimental.pallas import tpu_sc as plsc

B = 1024
T = 200
D = 64
BT = B * T
NC = 2
NS = 16
NW = NC * NS
B_PER_W = BT // NW
CHUNK = 64
N_CHUNKS = B_PER_W // CHUNK
NBUF = 4
LANES = 16
GROUPS = D // LANES
POS_PAD = T + CHUNK

_mesh = plsc.VectorSubcoreMesh(core_axis_name="c", subcore_axis_name="s")


@functools.partial(
    pl.kernel,
    mesh=_mesh,
    out_type=jax.ShapeDtypeStruct((BT, D), jnp.float32),
    scratch_types=[
        pltpu.VMEM((N_CHUNKS, CHUNK), jnp.int32),
        pltpu.VMEM((NBUF, CHUNK, D), jnp.float32),
        pltpu.VMEM((POS_PAD, D), jnp.float32),
    ] + [pltpu.SemaphoreType.DMA] * (2 * NBUF),
    compiler_params=pltpu.CompilerParams(use_tc_tiling_on_sc=False),
)
def _embed_sc(x_hbm, tok_hbm, pos_hbm, out_hbm, idx_v, rows_v, pos_v,
              g0, g1, g2, g3, o0, o1, o2, o3):
    gsems = (g0, g1, g2, g3)
    osems = (o0, o1, o2, o3)
    cid = lax.axis_index("c")
    sid = lax.axis_index("s")
    wid = sid * NC + cid
    base = wid * B_PER_W

    pltpu.sync_copy(x_hbm.at[wid], idx_v)
    pltpu.sync_copy(pos_hbm.at[pl.ds(0, T)], pos_v.at[pl.ds(0, T)])
    pltpu.sync_copy(pos_hbm.at[pl.ds(0, CHUNK)], pos_v.at[pl.ds(T, CHUNK)])

    def gather(ch, b):
        pltpu.async_copy(tok_hbm.at[idx_v.at[ch]], rows_v.at[b], gsems[b])

    def wait_gather(ch, b):
        pltpu.make_async_copy(
            tok_hbm.at[idx_v.at[ch]], rows_v.at[b], gsems[b]).wait()

    def put(ch, b):
        pltpu.async_copy(
            rows_v.at[b], out_hbm.at[pl.ds(base + ch * CHUNK, CHUNK)],
            osems[b])

    def wait_put(b):
        pltpu.make_async_copy(
            rows_v.at[b], out_hbm.at[pl.ds(base, CHUNK)], osems[b]).wait()

    for b in range(NBUF - 1):
        gather(b, b)

    def block_body(blk, carry):
        c0 = blk * NBUF
        for b in range(NBUF):
            ch = c0 + b
            wait_gather(ch, b)
            p0 = lax.rem(ch * CHUNK, T)

            @plsc.parallel_loop(0, CHUNK, unroll=8)
            def _row(r):
                for g in range(GROUPS):
                    sl = pl.ds(g * LANES, LANES)
                    rows_v[b, r, sl] = rows_v[b, r, sl] + pos_v[p0 + r, sl]

            put(ch, b)
            nxt = ch + NBUF - 1
            bn = (b + NBUF - 1) % NBUF

            @pl.when(nxt < N_CHUNKS)
            def _():
                @pl.when(nxt >= NBUF)
                def _():
                    wait_put(bn)
                gather(nxt, bn)
        return carry

    lax.fori_loop(0, N_CHUNKS // NBUF, block_body, 0)

    for b in range(NBUF):
        wait_put(b)


def kernel(x, token_emb, pos_emb):
    xw = x.reshape(NW, N_CHUNKS, CHUNK).astype(jnp.int32)
    out = _embed_sc(xw, token_emb, pos_emb)
    return out.reshape(B, T, D)
